# Initial kernel scaffold; baseline (speedup 1.0000x reference)
import jax
import jax.numpy as jnp
from jax.experimental import pallas as pl


def kernel(feat, coordinate, edge_index, Win, b_in, Wout, b_out, We1, be1, We2, be2, Wc1, bc1, Wc2, bc2, Wn1, bn1, Wn2, bn2):
    raise NotImplementedError("write your pallas kernel here")


# speedup vs baseline: 2.1590x; 2.1590x over previous
import functools

import jax
import jax.numpy as jnp
from jax import lax
from jax.experimental import pallas as pl
from jax.experimental.pallas import tpu as pltpu
from jax.experimental.pallas import tpu_sc as plsc

F32 = jnp.float32

_NC = 2
_NS = 16
_NW = _NC * _NS
_K = 40
_XP = 16
_XG = 128
_BE = 1000
_BN = 1000


def _silu(v):
    return v * jax.nn.sigmoid(v)


def _mmb_body(x_ref, w_ref, b_ref, o_ref):
    o_ref[...] = (
        jnp.dot(x_ref[...], w_ref[...], preferred_element_type=F32) + b_ref[...]
    )


def _mmb(x, w, b):
    n, din = x.shape
    dout = w.shape[1]
    return pl.pallas_call(
        _mmb_body,
        grid=(n // _BN,),
        in_specs=[
            pl.BlockSpec((_BN, din), lambda i: (i, 0)),
            pl.BlockSpec((din, dout), lambda i: (0, 0)),
            pl.BlockSpec((1, dout), lambda i: (0, 0)),
        ],
        out_specs=pl.BlockSpec((_BN, dout), lambda i: (i, 0)),
        out_shape=jax.ShapeDtypeStruct((n, dout), F32),
    )(x, w, b.reshape(1, dout))


def _edge_body(hd_ref, hs_ref, xd_ref, xs_ref, w1a_ref, w1b_ref, w1r_ref,
               b1_ref, w2_ref, b2_ref, wc1_ref, bc1_ref, wc2_ref, bc2_ref,
               m_ref, mc_ref):
    bf = lambda v: v.astype(jnp.bfloat16).astype(F32)
    diff = xd_ref[...] - xs_ref[...]
    r2 = jnp.sum(diff * diff, axis=1, keepdims=True)
    t = (jnp.dot(hd_ref[...], w1a_ref[...], preferred_element_type=F32)
         + jnp.dot(hs_ref[...], w1b_ref[...], preferred_element_type=F32)
         + bf(r2) * bf(w1r_ref[...]) + b1_ref[...])
    t = _silu(t)
    m = _silu(jnp.dot(t, w2_ref[...], preferred_element_type=F32) + b2_ref[...])
    u = _silu(jnp.dot(m, wc1_ref[...], preferred_element_type=F32) + bc1_ref[...])
    cw = jnp.sum(bf(u) * bf(wc2_ref[...]), axis=1, keepdims=True) + bc2_ref[...]
    m_ref[...] = m
    mc_ref[...] = diff * cw


def _edge_tc(hd, hs, xd, xs, w1a, w1b, w1r, b1, w2, b2, wc1, bc1, wc2, bc2):
    e, h = hd.shape
    full = lambda r, c: pl.BlockSpec((r, c), lambda i: (0, 0))
    return pl.pallas_call(
        _edge_body,
        grid=(e // _BE,),
        in_specs=[
            pl.BlockSpec((_BE, h), lambda i: (i, 0)),
            pl.BlockSpec((_BE, h), lambda i: (i, 0)),
            pl.BlockSpec((_BE, _XG), lambda i: (i, 0)),
            pl.BlockSpec((_BE, _XG), lambda i: (i, 0)),
            full(h, h), full(h, h), full(1, h), full(1, h),
            full(h, h), full(1, h), full(h, h), full(1, h),
            full(1, h), full(1, 1),
        ],
        out_specs=[
            pl.BlockSpec((_BE, h), lambda i: (i, 0)),
            pl.BlockSpec((_BE, _XG), lambda i: (i, 0)),
        ],
        out_shape=[
            jax.ShapeDtypeStruct((e, h), F32),
            jax.ShapeDtypeStruct((e, _XG), F32),
        ],
    )(hd, hs, xd, xs, w1a, w1b, w1r, b1, w2, b2, wc1, bc1, wc2, bc2)


def _node_body(h_ref, am_ref, ax_ref, x_ref, wn1a_ref, wn1b_ref, bn1_ref,
               wn2_ref, bn2_ref, ho_ref, xo_ref):
    aggm = am_ref[0] + am_ref[1]
    aggx = (ax_ref[0] + ax_ref[1])[:, :_XP]
    t = (jnp.dot(h_ref[...], wn1a_ref[...], preferred_element_type=F32)
         + jnp.dot(aggm, wn1b_ref[...], preferred_element_type=F32)
         + bn1_ref[...])
    t = _silu(t)
    ho_ref[...] = (h_ref[...]
                   + jnp.dot(t, wn2_ref[...], preferred_element_type=F32)
                   + bn2_ref[...])
    xo_ref[...] = x_ref[...] + aggx * 0.1


def _node_tc(h, am2, ax2, xpad, wn1a, wn1b, bn1, wn2, bn2):
    n, hd = h.shape
    full = lambda r, c: pl.BlockSpec((r, c), lambda i: (0, 0))
    return pl.pallas_call(
        _node_body,
        grid=(n // _BN,),
        in_specs=[
            pl.BlockSpec((_BN, hd), lambda i: (i, 0)),
            pl.BlockSpec((2, _BN, hd), lambda i: (0, i, 0)),
            pl.BlockSpec((2, _BN, _XG), lambda i: (0, i, 0)),
            pl.BlockSpec((_BN, _XP), lambda i: (i, 0)),
            full(hd, hd), full(hd, hd), full(1, hd), full(hd, hd), full(1, hd),
        ],
        out_specs=[
            pl.BlockSpec((_BN, hd), lambda i: (i, 0)),
            pl.BlockSpec((_BN, _XP), lambda i: (i, 0)),
        ],
        out_shape=[
            jax.ShapeDtypeStruct((n, hd), F32),
            jax.ShapeDtypeStruct((n, _XP), F32),
        ],
    )(h, am2, ax2, xpad, wn1a, wn1b, bn1, wn2, bn2)


def _sc_gather(h, xpad, src2, dst2):
    n, hd = h.shape
    nch = src2.shape[1]
    e = _NW * nch * _K
    mesh = plsc.VectorSubcoreMesh(core_axis_name="c", subcore_axis_name="s")

    @functools.partial(
        pl.kernel,
        out_type=(
            jax.ShapeDtypeStruct((e, hd), F32),
            jax.ShapeDtypeStruct((e, hd), F32),
            jax.ShapeDtypeStruct((e, _XG), F32),
            jax.ShapeDtypeStruct((e, _XG), F32),
        ),
        mesh=mesh,
        scratch_types=[
            pltpu.VMEM((nch, _K), jnp.int32),
            pltpu.VMEM((nch, _K), jnp.int32),
            pltpu.VMEM((_K, hd), F32),
            pltpu.VMEM((_K, hd), F32),
            pltpu.VMEM((_K, _XG), F32),
            pltpu.VMEM((_K, _XG), F32),
            pltpu.SemaphoreType.DMA,
            pltpu.SemaphoreType.DMA,
        ],
    )
    def k(h_hbm, x_hbm, src_hbm, dst_hbm, hd_hbm, hs_hbm, xd_hbm, xs_hbm,
          sidx, didx, hdb, hsb, xdb, xsb, semg, semw):
        c = lax.axis_index("c")
        s = lax.axis_index("s")
        wid = s * _NC + c
        pltpu.sync_copy(dst_hbm.at[wid], didx)
        pltpu.sync_copy(src_hbm.at[wid], sidx)

        def chunk(j, carry):
            off = (wid * nch + j) * _K
            g1 = pltpu.make_async_copy(h_hbm.at[didx.at[j]], hdb, semg)
            g2 = pltpu.make_async_copy(h_hbm.at[sidx.at[j]], hsb, semg)
            g3 = pltpu.make_async_copy(x_hbm.at[didx.at[j]], xdb, semg)
            g4 = pltpu.make_async_copy(x_hbm.at[sidx.at[j]], xsb, semg)
            g1.start(); g2.start(); g3.start(); g4.start()
            g1.wait(); g2.wait(); g3.wait(); g4.wait()
            w1 = pltpu.make_async_copy(hdb, hd_hbm.at[pl.ds(off, _K)], semw)
            w2 = pltpu.make_async_copy(hsb, hs_hbm.at[pl.ds(off, _K)], semw)
            w3 = pltpu.make_async_copy(xdb, xd_hbm.at[pl.ds(off, _K)], semw)
            w4 = pltpu.make_async_copy(xsb, xs_hbm.at[pl.ds(off, _K)], semw)
            w1.start(); w2.start(); w3.start(); w4.start()
            w1.wait(); w2.wait(); w3.wait(); w4.wait()
            return carry

        lax.fori_loop(0, nch, chunk, 0)

    return k(h, xpad, src2, dst2)


def _sc_scatter(vals, dst2, zrows):
    e, dv = vals.shape
    d = zrows.shape[1]
    nch = dst2.shape[1]
    rpt = zrows.shape[0]
    ndr = 10
    n = rpt * ndr
    mesh = plsc.VectorSubcoreMesh(core_axis_name="c", subcore_axis_name="s")

    @functools.partial(
        pl.kernel,
        out_type=jax.ShapeDtypeStruct((_NC, n, d), F32),
        mesh=mesh,
        scratch_types=[
            pltpu.VMEM((nch, _K), jnp.int32),
            pltpu.VMEM((_K, d), F32),
            pltpu.VMEM_SHARED((n, d), F32),
        ],
    )
    def k(v_hbm, dst_hbm, z_hbm, acc_hbm, didx, vb, acc):
        c = lax.axis_index("c")
        s = lax.axis_index("s")
        wid = s * _NC + c

        @pl.when(s < ndr)
        def _init():
            pltpu.sync_copy(z_hbm, acc.at[pl.ds(s * rpt, rpt)])

        if dv < d:
            pltpu.sync_copy(z_hbm.at[pl.ds(0, _K)], vb)
        plsc.subcore_barrier()
        pltpu.sync_copy(dst_hbm.at[wid], didx)

        def chunk(j, carry):
            off = (wid * nch + j) * _K
            if dv < d:
                pltpu.sync_copy(v_hbm.at[pl.ds(off, _K)],
                                vb.at[:, pl.ds(0, dv)])
            else:
                pltpu.sync_copy(v_hbm.at[pl.ds(off, _K)], vb)
            pltpu.sync_copy(vb, acc.at[didx.at[j]], add=True)
            return carry

        lax.fori_loop(0, nch, chunk, 0)
        plsc.subcore_barrier()

        @pl.when(s < ndr)
        def _drain():
            pltpu.sync_copy(acc.at[pl.ds(s * rpt, rpt)],
                            acc_hbm.at[c, pl.ds(s * rpt, rpt)])

    return k(vals, dst2, zrows)


def kernel(feat, coordinate, edge_index, Win, b_in, Wout, b_out,
           We1, be1, We2, be2, Wc1, bc1, Wc2, bc2, Wn1, bn1, Wn2, bn2):
    n, _ = feat.shape
    e = edge_index.shape[1]
    h_dim = Win.shape[1]
    depth = We1.shape[0]

    src2 = edge_index[0].reshape(_NW, e // (_NW * _K), _K)
    dst2 = edge_index[1].reshape(_NW, e // (_NW * _K), _K)
    xpad = jnp.pad(coordinate, ((0, 0), (0, _XP - coordinate.shape[1])))
    z128 = jnp.zeros((n // 10, _XG), F32)

    h = _mmb(feat, Win, b_in)
    for l in range(depth):
        xg = jnp.pad(xpad, ((0, 0), (0, _XG - _XP)))
        hd, hs, xd, xs = _sc_gather(h, xg, src2, dst2)
        m, mc = _edge_tc(
            hd, hs, xd, xs,
            We1[l, :h_dim], We1[l, h_dim:2 * h_dim], We1[l, 2 * h_dim:],
            be1[l].reshape(1, -1), We2[l], be2[l].reshape(1, -1),
            Wc1[l], bc1[l].reshape(1, -1), Wc2[l].T, bc2[l].reshape(1, 1),
        )
        am2 = _sc_scatter(m, dst2, z128)
        ax2 = _sc_scatter(mc, dst2, z128)
        h, xpad = _node_tc(
            h, am2, ax2, xpad,
            Wn1[l, :h_dim], Wn1[l, h_dim:], bn1[l].reshape(1, -1),
            Wn2[l], bn2[l].reshape(1, -1),
        )
    out = _mmb(h, Wout, b_out)
    return (out, xpad[:, :coordinate.shape[1]])


# speedup vs baseline: 2.4219x; 1.1218x over previous
import functools

import jax
import jax.numpy as jnp
from jax import lax
from jax.experimental import pallas as pl
from jax.experimental.pallas import tpu as pltpu
from jax.experimental.pallas import tpu_sc as plsc

F32 = jnp.float32

_NC = 2
_NS = 16
_NW = _NC * _NS
_K = 40
_XP = 16
_XG = 128
_BE = 1000
_BN = 1000


def _silu(v):
    return v * jax.nn.sigmoid(v)


def _pack(hval, xval):
    b = lax.bitcast_convert_type(
        hval.astype(jnp.bfloat16).astype(F32), jnp.uint32)
    w = (b[:, :64] >> 16) | (b[:, 64:] & jnp.uint32(0xFFFF0000))
    xw = lax.bitcast_convert_type(xval, jnp.uint32)
    pad = jnp.zeros((hval.shape[0], 48), jnp.uint32)
    packed = jnp.concatenate([w, xw, pad], axis=1)
    return lax.bitcast_convert_type(packed, jnp.int32)


def _unpack(tval):
    v = lax.bitcast_convert_type(tval, jnp.uint32)
    hw = v[:, :64]
    lo = lax.bitcast_convert_type(hw << 16, F32)
    hi = lax.bitcast_convert_type(hw & jnp.uint32(0xFFFF0000), F32)
    h = jnp.concatenate([lo, hi], axis=1)
    x = lax.bitcast_convert_type(v[:, 64:80], F32)
    return h, x


def _mmb_body(x_ref, w_ref, b_ref, o_ref):
    o_ref[...] = (
        jnp.dot(x_ref[...], w_ref[...], preferred_element_type=F32) + b_ref[...]
    )


def _mmb(x, w, b):
    n, din = x.shape
    dout = w.shape[1]
    return pl.pallas_call(
        _mmb_body,
        grid=(n // _BN,),
        in_specs=[
            pl.BlockSpec((_BN, din), lambda i: (i, 0)),
            pl.BlockSpec((din, dout), lambda i: (0, 0)),
            pl.BlockSpec((1, dout), lambda i: (0, 0)),
        ],
        out_specs=pl.BlockSpec((_BN, dout), lambda i: (i, 0)),
        out_shape=jax.ShapeDtypeStruct((n, dout), F32),
    )(x, w, b.reshape(1, dout))


def _embed_body(x_ref, w_ref, b_ref, xp_ref, o_ref, t_ref):
    h = jnp.dot(x_ref[...], w_ref[...], preferred_element_type=F32) + b_ref[...]
    o_ref[...] = h
    t_ref[...] = _pack(h, xp_ref[...])


def _embed(x, w, b, xpad):
    n, din = x.shape
    dout = w.shape[1]
    return pl.pallas_call(
        _embed_body,
        grid=(n // _BN,),
        in_specs=[
            pl.BlockSpec((_BN, din), lambda i: (i, 0)),
            pl.BlockSpec((din, dout), lambda i: (0, 0)),
            pl.BlockSpec((1, dout), lambda i: (0, 0)),
            pl.BlockSpec((_BN, _XP), lambda i: (i, 0)),
        ],
        out_specs=[
            pl.BlockSpec((_BN, dout), lambda i: (i, 0)),
            pl.BlockSpec((_BN, _XG), lambda i: (i, 0)),
        ],
        out_shape=[
            jax.ShapeDtypeStruct((n, dout), F32),
            jax.ShapeDtypeStruct((n, _XG), jnp.int32),
        ],
    )(x, w, b.reshape(1, dout), xpad)


def _edge_body(td_ref, ts_ref, w1a_ref, w1b_ref, w1r_ref,
               b1_ref, w2_ref, b2_ref, wc1_ref, bc1_ref, wc2_ref, bc2_ref,
               m_ref, mc_ref):
    bf = lambda v: v.astype(jnp.bfloat16).astype(F32)
    hd, xd = _unpack(td_ref[...])
    hs, xs = _unpack(ts_ref[...])
    diff = xd - xs
    r2 = jnp.sum(diff * diff, axis=1, keepdims=True)
    t = (jnp.dot(hd, w1a_ref[...], preferred_element_type=F32)
         + jnp.dot(hs, w1b_ref[...], preferred_element_type=F32)
         + bf(r2) * bf(w1r_ref[...]) + b1_ref[...])
    t = _silu(t)
    m = _silu(jnp.dot(t, w2_ref[...], preferred_element_type=F32) + b2_ref[...])
    u = _silu(jnp.dot(m, wc1_ref[...], preferred_element_type=F32) + bc1_ref[...])
    cw = jnp.sum(bf(u) * bf(wc2_ref[...]), axis=1, keepdims=True) + bc2_ref[...]
    m_ref[...] = m
    mc = diff * cw
    mc_ref[...] = jnp.concatenate(
        [mc, jnp.zeros((mc.shape[0], _XG - _XP), F32)], axis=1)


def _edge_tc(td, ts, w1a, w1b, w1r, b1, w2, b2, wc1, bc1, wc2, bc2):
    e = td.shape[0]
    h = w1a.shape[1]
    full = lambda r, c: pl.BlockSpec((r, c), lambda i: (0, 0))
    return pl.pallas_call(
        _edge_body,
        grid=(e // _BE,),
        in_specs=[
            pl.BlockSpec((_BE, _XG), lambda i: (i, 0)),
            pl.BlockSpec((_BE, _XG), lambda i: (i, 0)),
            full(h, h), full(h, h), full(1, h), full(1, h),
            full(h, h), full(1, h), full(h, h), full(1, h),
            full(1, h), full(1, 1),
        ],
        out_specs=[
            pl.BlockSpec((_BE, h), lambda i: (i, 0)),
            pl.BlockSpec((_BE, _XG), lambda i: (i, 0)),
        ],
        out_shape=[
            jax.ShapeDtypeStruct((e, h), F32),
            jax.ShapeDtypeStruct((e, _XG), F32),
        ],
    )(td, ts, w1a, w1b, w1r, b1, w2, b2, wc1, bc1, wc2, bc2)


def _node_body(h_ref, am_ref, ax_ref, x_ref, wn1a_ref, wn1b_ref, bn1_ref,
               wn2_ref, bn2_ref, ho_ref, xo_ref, to_ref):
    aggm = am_ref[0] + am_ref[1]
    aggx = (ax_ref[0] + ax_ref[1])[:, :_XP]
    t = (jnp.dot(h_ref[...], wn1a_ref[...], preferred_element_type=F32)
         + jnp.dot(aggm, wn1b_ref[...], preferred_element_type=F32)
         + bn1_ref[...])
    t = _silu(t)
    ho = (h_ref[...]
          + jnp.dot(t, wn2_ref[...], preferred_element_type=F32)
          + bn2_ref[...])
    xo = x_ref[...] + aggx * 0.1
    ho_ref[...] = ho
    xo_ref[...] = xo
    to_ref[...] = _pack(ho, xo)


def _node_tc(h, am2, ax2, xpad, wn1a, wn1b, bn1, wn2, bn2):
    n, hd = h.shape
    full = lambda r, c: pl.BlockSpec((r, c), lambda i: (0, 0))
    return pl.pallas_call(
        _node_body,
        grid=(n // _BN,),
        in_specs=[
            pl.BlockSpec((_BN, hd), lambda i: (i, 0)),
            pl.BlockSpec((2, _BN, hd), lambda i: (0, i, 0)),
            pl.BlockSpec((2, _BN, _XG), lambda i: (0, i, 0)),
            pl.BlockSpec((_BN, _XP), lambda i: (i, 0)),
            full(hd, hd), full(hd, hd), full(1, hd), full(hd, hd), full(1, hd),
        ],
        out_specs=[
            pl.BlockSpec((_BN, hd), lambda i: (i, 0)),
            pl.BlockSpec((_BN, _XP), lambda i: (i, 0)),
            pl.BlockSpec((_BN, _XG), lambda i: (i, 0)),
        ],
        out_shape=[
            jax.ShapeDtypeStruct((n, hd), F32),
            jax.ShapeDtypeStruct((n, _XP), F32),
            jax.ShapeDtypeStruct((n, _XG), jnp.int32),
        ],
    )(h, am2, ax2, xpad, wn1a, wn1b, bn1, wn2, bn2)


def _sc_gather(tbl, src2, dst2):
    n, d = tbl.shape
    nch = src2.shape[1]
    e = _NW * nch * _K
    mesh = plsc.VectorSubcoreMesh(core_axis_name="c", subcore_axis_name="s")

    @functools.partial(
        pl.kernel,
        out_type=(
            jax.ShapeDtypeStruct((e, d), jnp.int32),
            jax.ShapeDtypeStruct((e, d), jnp.int32),
        ),
        mesh=mesh,
        scratch_types=[
            pltpu.VMEM((nch, _K), jnp.int32),
            pltpu.VMEM((nch, _K), jnp.int32),
            pltpu.VMEM((2, _K, _XG), jnp.int32),
            pltpu.VMEM((2, _K, _XG), jnp.int32),
            pltpu.SemaphoreType.DMA,
            pltpu.SemaphoreType.DMA,
            pltpu.SemaphoreType.DMA,
            pltpu.SemaphoreType.DMA,
        ],
    )
    def k(t_hbm, src_hbm, dst_hbm, td_hbm, ts_hbm,
          sidx, didx, tdb, tsb, semg0, semg1, semw0, semw1):
        c = lax.axis_index("c")
        s = lax.axis_index("s")
        wid = s * _NC + c
        semg = (semg0, semg1)
        semw = (semw0, semw1)
        pltpu.sync_copy(dst_hbm.at[wid], didx)
        pltpu.sync_copy(src_hbm.at[wid], sidx)

        def gathers(j, b, start):
            g1 = pltpu.make_async_copy(t_hbm.at[didx.at[j]], tdb.at[b],
                                       semg[b])
            g2 = pltpu.make_async_copy(t_hbm.at[sidx.at[j]], tsb.at[b],
                                       semg[b])
            if start:
                g1.start(); g2.start()
            else:
                g1.wait(); g2.wait()

        def writes(j, b, start):
            off = (wid * nch + j) * _K
            w1 = pltpu.make_async_copy(tdb.at[b], td_hbm.at[pl.ds(off, _K)],
                                       semw[b])
            w2 = pltpu.make_async_copy(tsb.at[b], ts_hbm.at[pl.ds(off, _K)],
                                       semw[b])
            if start:
                w1.start(); w2.start()
            else:
                w1.wait(); w2.wait()

        gathers(0, 0, True)

        def loop(i, carry):
            for b in (0, 1):
                j = 2 * i + b
                gathers(j, b, False)

                @pl.when(j + 1 < nch)
                def _next():
                    @pl.when(j >= 1)
                    def _drain():
                        writes(j - 1, 1 - b, False)
                    gathers(j + 1, 1 - b, True)

                writes(j, b, True)
            return carry

        lax.fori_loop(0, nch // 2, loop, 0)
        writes(nch - 2, nch % 2, False)
        writes(nch - 1, (nch - 1) % 2, False)

    return k(tbl, src2, dst2)


def _sc_scatter(vals, dst2, zrows):
    e, dv = vals.shape
    d = zrows.shape[1]
    nch = dst2.shape[1]
    rpt = zrows.shape[0]
    ndr = 10
    n = rpt * ndr
    mesh = plsc.VectorSubcoreMesh(core_axis_name="c", subcore_axis_name="s")

    @functools.partial(
        pl.kernel,
        out_type=jax.ShapeDtypeStruct((_NC, n, d), F32),
        mesh=mesh,
        scratch_types=[
            pltpu.VMEM((nch, _K), jnp.int32),
            pltpu.VMEM((_K, d), F32),
            pltpu.VMEM_SHARED((n, d), F32),
        ],
    )
    def k(v_hbm, dst_hbm, z_hbm, acc_hbm, didx, vb, acc):
        c = lax.axis_index("c")
        s = lax.axis_index("s")
        wid = s * _NC + c

        @pl.when(s < ndr)
        def _init():
            pltpu.sync_copy(z_hbm, acc.at[pl.ds(s * rpt, rpt)])

        if dv < d:
            pltpu.sync_copy(z_hbm.at[pl.ds(0, _K)], vb)
        plsc.subcore_barrier()
        pltpu.sync_copy(dst_hbm.at[wid], didx)

        def chunk(j, carry):
            off = (wid * nch + j) * _K
            if dv < d:
                pltpu.sync_copy(v_hbm.at[pl.ds(off, _K)],
                                vb.at[:, pl.ds(0, dv)])
            else:
                pltpu.sync_copy(v_hbm.at[pl.ds(off, _K)], vb)
            pltpu.sync_copy(vb, acc.at[didx.at[j]], add=True)
            return carry

        lax.fori_loop(0, nch, chunk, 0)
        plsc.subcore_barrier()

        @pl.when(s < ndr)
        def _drain():
            pltpu.sync_copy(acc.at[pl.ds(s * rpt, rpt)],
                            acc_hbm.at[c, pl.ds(s * rpt, rpt)])

    return k(vals, dst2, zrows)


def kernel(feat, coordinate, edge_index, Win, b_in, Wout, b_out,
           We1, be1, We2, be2, Wc1, bc1, Wc2, bc2, Wn1, bn1, Wn2, bn2):
    n, _ = feat.shape
    e = edge_index.shape[1]
    h_dim = Win.shape[1]
    depth = We1.shape[0]

    src2 = edge_index[0].reshape(_NW, e // (_NW * _K), _K)
    dst2 = edge_index[1].reshape(_NW, e // (_NW * _K), _K)
    xpad = jnp.pad(coordinate, ((0, 0), (0, _XP - coordinate.shape[1])))
    z128 = jnp.zeros((n // 10, _XG), F32)

    h, tbl = _embed(feat, Win, b_in, xpad)
    for l in range(depth):
        td, ts = _sc_gather(tbl, src2, dst2)
        m, mc = _edge_tc(
            td, ts,
            We1[l, :h_dim], We1[l, h_dim:2 * h_dim], We1[l, 2 * h_dim:],
            be1[l].reshape(1, -1), We2[l], be2[l].reshape(1, -1),
            Wc1[l], bc1[l].reshape(1, -1), Wc2[l].T, bc2[l].reshape(1, 1),
        )
        am2 = _sc_scatter(m, dst2, z128)
        ax2 = _sc_scatter(mc, dst2, z128)
        h, xpad, tbl = _node_tc(
            h, am2, ax2, xpad,
            Wn1[l, :h_dim], Wn1[l, h_dim:], bn1[l].reshape(1, -1),
            Wn2[l], bn2[l].reshape(1, -1),
        )
    out = _mmb(h, Wout, b_out)
    return (out, xpad[:, :coordinate.shape[1]])


# speedup vs baseline: 3.0008x; 1.2390x over previous
import functools

import jax
import jax.numpy as jnp
from jax import lax
from jax.experimental import pallas as pl
from jax.experimental.pallas import tpu as pltpu
from jax.experimental.pallas import tpu_sc as plsc

F32 = jnp.float32

_NC = 2
_NS = 16
_NW = _NC * _NS
_K = 40
_XP = 16
_XG = 128
_BE = 2000
_BN = 1000


def _silu(v):
    return v * jax.nn.sigmoid(v)


def _pack(hval, xval):
    b = lax.bitcast_convert_type(
        hval.astype(jnp.bfloat16).astype(F32), jnp.uint32)
    w = (b[:, :64] >> 16) | (b[:, 64:] & jnp.uint32(0xFFFF0000))
    xw = lax.bitcast_convert_type(xval, jnp.uint32)
    pad = jnp.zeros((hval.shape[0], 48), jnp.uint32)
    packed = jnp.concatenate([w, xw, pad], axis=1)
    return lax.bitcast_convert_type(packed, jnp.int32)


def _unpack(tval):
    v = lax.bitcast_convert_type(tval, jnp.uint32)
    hw = v[:, :64]
    lo = lax.bitcast_convert_type(hw << 16, F32)
    hi = lax.bitcast_convert_type(hw & jnp.uint32(0xFFFF0000), F32)
    h = jnp.concatenate([lo, hi], axis=1)
    x = lax.bitcast_convert_type(v[:, 64:80], F32)
    return h, x


def _mmb_body(x_ref, w_ref, b_ref, o_ref):
    o_ref[...] = (
        jnp.dot(x_ref[...], w_ref[...], preferred_element_type=F32) + b_ref[...]
    )


def _mmb(x, w, b):
    n, din = x.shape
    dout = w.shape[1]
    return pl.pallas_call(
        _mmb_body,
        grid=(n // _BN,),
        in_specs=[
            pl.BlockSpec((_BN, din), lambda i: (i, 0)),
            pl.BlockSpec((din, dout), lambda i: (0, 0)),
            pl.BlockSpec((1, dout), lambda i: (0, 0)),
        ],
        out_specs=pl.BlockSpec((_BN, dout), lambda i: (i, 0)),
        out_shape=jax.ShapeDtypeStruct((n, dout), F32),
    )(x, w, b.reshape(1, dout))


def _embed_body(x_ref, w_ref, b_ref, xp_ref, o_ref, t_ref):
    h = jnp.dot(x_ref[...], w_ref[...], preferred_element_type=F32) + b_ref[...]
    o_ref[...] = h
    t_ref[...] = _pack(h, xp_ref[...])


def _embed(x, w, b, xpad):
    n, din = x.shape
    dout = w.shape[1]
    return pl.pallas_call(
        _embed_body,
        grid=(n // _BN,),
        in_specs=[
            pl.BlockSpec((_BN, din), lambda i: (i, 0)),
            pl.BlockSpec((din, dout), lambda i: (0, 0)),
            pl.BlockSpec((1, dout), lambda i: (0, 0)),
            pl.BlockSpec((_BN, _XP), lambda i: (i, 0)),
        ],
        out_specs=[
            pl.BlockSpec((_BN, dout), lambda i: (i, 0)),
            pl.BlockSpec((_BN, _XG), lambda i: (i, 0)),
        ],
        out_shape=[
            jax.ShapeDtypeStruct((n, dout), F32),
            jax.ShapeDtypeStruct((n, _XG), jnp.int32),
        ],
    )(x, w, b.reshape(1, dout), xpad)


def _edge_body(td_ref, ts_ref, w1a_ref, w1b_ref, w1r_ref,
               b1_ref, w2_ref, b2_ref, wc1_ref, bc1_ref, wc2_ref, bc2_ref,
               m_ref, mc_ref):
    bf = lambda v: v.astype(jnp.bfloat16).astype(F32)
    hd, xd = _unpack(td_ref[...])
    hs, xs = _unpack(ts_ref[...])
    diff = xd - xs
    r2 = jnp.sum(diff * diff, axis=1, keepdims=True)
    t = (jnp.dot(hd, w1a_ref[...], preferred_element_type=F32)
         + jnp.dot(hs, w1b_ref[...], preferred_element_type=F32)
         + bf(r2) * bf(w1r_ref[...]) + b1_ref[...])
    t = _silu(t)
    m = _silu(jnp.dot(t, w2_ref[...], preferred_element_type=F32) + b2_ref[...])
    u = _silu(jnp.dot(m, wc1_ref[...], preferred_element_type=F32) + bc1_ref[...])
    cw = jnp.sum(bf(u) * bf(wc2_ref[...]), axis=1, keepdims=True) + bc2_ref[...]
    m_ref[...] = m
    mc_ref[...] = diff * cw


def _edge_tc(td, ts, w1a, w1b, w1r, b1, w2, b2, wc1, bc1, wc2, bc2):
    e = td.shape[0]
    h = w1a.shape[1]
    full = lambda r, c: pl.BlockSpec((r, c), lambda i: (0, 0))
    return pl.pallas_call(
        _edge_body,
        grid=(e // _BE,),
        in_specs=[
            pl.BlockSpec((_BE, _XG), lambda i: (i, 0)),
            pl.BlockSpec((_BE, _XG), lambda i: (i, 0)),
            full(h, h), full(h, h), full(1, h), full(1, h),
            full(h, h), full(1, h), full(h, h), full(1, h),
            full(1, h), full(1, 1),
        ],
        out_specs=[
            pl.BlockSpec((_BE, h), lambda i: (i, 0)),
            pl.BlockSpec((_BE, _XP), lambda i: (i, 0)),
        ],
        out_shape=[
            jax.ShapeDtypeStruct((e, h), F32),
            jax.ShapeDtypeStruct((e, _XP), F32),
        ],
    )(td, ts, w1a, w1b, w1r, b1, w2, b2, wc1, bc1, wc2, bc2)


def _node_body(h_ref, am_ref, ax_ref, x_ref, wn1a_ref, wn1b_ref, bn1_ref,
               wn2_ref, bn2_ref, ho_ref, xo_ref, to_ref):
    aggm = am_ref[0] + am_ref[1]
    aggx = (ax_ref[0] + ax_ref[1])[:, :_XP]
    t = (jnp.dot(h_ref[...], wn1a_ref[...], preferred_element_type=F32)
         + jnp.dot(aggm, wn1b_ref[...], preferred_element_type=F32)
         + bn1_ref[...])
    t = _silu(t)
    ho = (h_ref[...]
          + jnp.dot(t, wn2_ref[...], preferred_element_type=F32)
          + bn2_ref[...])
    xo = x_ref[...] + aggx * 0.1
    ho_ref[...] = ho
    xo_ref[...] = xo
    to_ref[...] = _pack(ho, xo)


def _node_tc(h, am2, ax2, xpad, wn1a, wn1b, bn1, wn2, bn2):
    n, hd = h.shape
    full = lambda r, c: pl.BlockSpec((r, c), lambda i: (0, 0))
    return pl.pallas_call(
        _node_body,
        grid=(n // _BN,),
        in_specs=[
            pl.BlockSpec((_BN, hd), lambda i: (i, 0)),
            pl.BlockSpec((2, _BN, hd), lambda i: (0, i, 0)),
            pl.BlockSpec((2, _BN, _XG), lambda i: (0, i, 0)),
            pl.BlockSpec((_BN, _XP), lambda i: (i, 0)),
            full(hd, hd), full(hd, hd), full(1, hd), full(hd, hd), full(1, hd),
        ],
        out_specs=[
            pl.BlockSpec((_BN, hd), lambda i: (i, 0)),
            pl.BlockSpec((_BN, _XP), lambda i: (i, 0)),
            pl.BlockSpec((_BN, _XG), lambda i: (i, 0)),
        ],
        out_shape=[
            jax.ShapeDtypeStruct((n, hd), F32),
            jax.ShapeDtypeStruct((n, _XP), F32),
            jax.ShapeDtypeStruct((n, _XG), jnp.int32),
        ],
    )(h, am2, ax2, xpad, wn1a, wn1b, bn1, wn2, bn2)


def _sc_gather(tbl, src2, dst2):
    n, d = tbl.shape
    nch = src2.shape[1]
    e = _NW * nch * _K
    mesh = plsc.VectorSubcoreMesh(core_axis_name="c", subcore_axis_name="s")

    @functools.partial(
        pl.kernel,
        out_type=(
            jax.ShapeDtypeStruct((e, d), jnp.int32),
            jax.ShapeDtypeStruct((e, d), jnp.int32),
        ),
        mesh=mesh,
        scratch_types=[
            pltpu.VMEM((nch, _K), jnp.int32),
            pltpu.VMEM((nch, _K), jnp.int32),
            pltpu.VMEM((2, _K, _XG), jnp.int32),
            pltpu.VMEM((2, _K, _XG), jnp.int32),
            pltpu.SemaphoreType.DMA,
            pltpu.SemaphoreType.DMA,
            pltpu.SemaphoreType.DMA,
            pltpu.SemaphoreType.DMA,
        ],
    )
    def k(t_hbm, src_hbm, dst_hbm, td_hbm, ts_hbm,
          sidx, didx, tdb, tsb, semg0, semg1, semw0, semw1):
        c = lax.axis_index("c")
        s = lax.axis_index("s")
        wid = s * _NC + c
        semg = (semg0, semg1)
        semw = (semw0, semw1)
        pltpu.sync_copy(dst_hbm.at[wid], didx)
        pltpu.sync_copy(src_hbm.at[wid], sidx)

        def gathers(j, b, start):
            g1 = pltpu.make_async_copy(t_hbm.at[didx.at[j]], tdb.at[b],
                                       semg[b])
            g2 = pltpu.make_async_copy(t_hbm.at[sidx.at[j]], tsb.at[b],
                                       semg[b])
            if start:
                g1.start(); g2.start()
            else:
                g1.wait(); g2.wait()

        def writes(j, b, start):
            off = (wid * nch + j) * _K
            w1 = pltpu.make_async_copy(tdb.at[b], td_hbm.at[pl.ds(off, _K)],
                                       semw[b])
            w2 = pltpu.make_async_copy(tsb.at[b], ts_hbm.at[pl.ds(off, _K)],
                                       semw[b])
            if start:
                w1.start(); w2.start()
            else:
                w1.wait(); w2.wait()

        gathers(0, 0, True)

        def loop(i, carry):
            for b in (0, 1):
                j = 2 * i + b
                gathers(j, b, False)

                @pl.when(j + 1 < nch)
                def _next():
                    @pl.when(j >= 1)
                    def _drain():
                        writes(j - 1, 1 - b, False)
                    gathers(j + 1, 1 - b, True)

                writes(j, b, True)
            return carry

        lax.fori_loop(0, nch // 2, loop, 0)
        writes(nch - 2, nch % 2, False)
        writes(nch - 1, (nch - 1) % 2, False)

    return k(tbl, src2, dst2)


def _sc_scatter(vals, dst2, zrows):
    e, dv = vals.shape
    d = zrows.shape[1]
    nch = dst2.shape[1]
    rpt = zrows.shape[0]
    ndr = 10
    n = rpt * ndr
    mesh = plsc.VectorSubcoreMesh(core_axis_name="c", subcore_axis_name="s")

    @functools.partial(
        pl.kernel,
        out_type=jax.ShapeDtypeStruct((_NC, n, d), F32),
        mesh=mesh,
        scratch_types=[
            pltpu.VMEM((nch, _K), jnp.int32),
            pltpu.VMEM((_K, d), F32),
            pltpu.VMEM((_K, _XP), F32),
            pltpu.VMEM_SHARED((n, d), F32),
        ],
    )
    def k(v_hbm, dst_hbm, z_hbm, acc_hbm, didx, vb, cb, acc):
        c = lax.axis_index("c")
        s = lax.axis_index("s")
        wid = s * _NC + c

        @pl.when(s < ndr)
        def _init():
            pltpu.sync_copy(z_hbm, acc.at[pl.ds(s * rpt, rpt)])

        if dv < d:
            pltpu.sync_copy(z_hbm.at[pl.ds(0, _K)], vb)
        plsc.subcore_barrier()
        pltpu.sync_copy(dst_hbm.at[wid], didx)

        def chunk(j, carry):
            off = (wid * nch + j) * _K
            if dv < d:
                pltpu.sync_copy(v_hbm.at[pl.ds(off, _K)], cb)
                for r in range(_K):
                    vb[r, pl.ds(0, dv)] = cb[r]
            else:
                pltpu.sync_copy(v_hbm.at[pl.ds(off, _K)], vb)
            pltpu.sync_copy(vb, acc.at[didx.at[j]], add=True)
            return carry

        lax.fori_loop(0, nch, chunk, 0)
        plsc.subcore_barrier()

        @pl.when(s < ndr)
        def _drain():
            pltpu.sync_copy(acc.at[pl.ds(s * rpt, rpt)],
                            acc_hbm.at[c, pl.ds(s * rpt, rpt)])

    return k(vals, dst2, zrows)


def kernel(feat, coordinate, edge_index, Win, b_in, Wout, b_out,
           We1, be1, We2, be2, Wc1, bc1, Wc2, bc2, Wn1, bn1, Wn2, bn2):
    n, _ = feat.shape
    e = edge_index.shape[1]
    h_dim = Win.shape[1]
    depth = We1.shape[0]

    src2 = edge_index[0].reshape(_NW, e // (_NW * _K), _K)
    dst2 = edge_index[1].reshape(_NW, e // (_NW * _K), _K)
    xpad = jnp.pad(coordinate, ((0, 0), (0, _XP - coordinate.shape[1])))
    z128 = jnp.zeros((n // 10, _XG), F32)

    h, tbl = _embed(feat, Win, b_in, xpad)
    for l in range(depth):
        td, ts = _sc_gather(tbl, src2, dst2)
        m, mc = _edge_tc(
            td, ts,
            We1[l, :h_dim], We1[l, h_dim:2 * h_dim], We1[l, 2 * h_dim:],
            be1[l].reshape(1, -1), We2[l], be2[l].reshape(1, -1),
            Wc1[l], bc1[l].reshape(1, -1), Wc2[l].T, bc2[l].reshape(1, 1),
        )
        am2 = _sc_scatter(m, dst2, z128)
        ax2 = _sc_scatter(mc, dst2, z128)
        h, xpad, tbl = _node_tc(
            h, am2, ax2, xpad,
            Wn1[l, :h_dim], Wn1[l, h_dim:], bn1[l].reshape(1, -1),
            Wn2[l], bn2[l].reshape(1, -1),
        )
    out = _mmb(h, Wout, b_out)
    return (out, xpad[:, :coordinate.shape[1]])


# speedup vs baseline: 3.4744x; 1.1578x over previous
import functools

import jax
import jax.numpy as jnp
from jax import lax
from jax.experimental import pallas as pl
from jax.experimental.pallas import tpu as pltpu
from jax.experimental.pallas import tpu_sc as plsc

F32 = jnp.float32

_NC = 2
_NS = 16
_NW = _NC * _NS
_K = 40
_XP = 16
_XG = 128
_BE = 1600
_BN = 1000


def _silu(v):
    return v * jax.nn.sigmoid(v)


def _pack(hval, xval):
    b = lax.bitcast_convert_type(
        hval.astype(jnp.bfloat16).astype(F32), jnp.uint32)
    w = (b[:, :64] >> 16) | (b[:, 64:] & jnp.uint32(0xFFFF0000))
    xw = lax.bitcast_convert_type(xval, jnp.uint32)
    pad = jnp.zeros((hval.shape[0], 48), jnp.uint32)
    packed = jnp.concatenate([w, xw, pad], axis=1)
    return lax.bitcast_convert_type(packed, jnp.int32)


def _unpack(tval):
    v = lax.bitcast_convert_type(tval, jnp.uint32)
    hw = v[:, :64]
    lo = lax.bitcast_convert_type(hw << 16, F32)
    hi = lax.bitcast_convert_type(hw & jnp.uint32(0xFFFF0000), F32)
    h = jnp.concatenate([lo, hi], axis=1)
    x = lax.bitcast_convert_type(v[:, 64:80], F32)
    return h, x


def _mmb_body(x_ref, w_ref, b_ref, o_ref):
    o_ref[...] = (
        jnp.dot(x_ref[...], w_ref[...], preferred_element_type=F32) + b_ref[...]
    )


def _mmb(x, w, b):
    n, din = x.shape
    dout = w.shape[1]
    return pl.pallas_call(
        _mmb_body,
        grid=(n // _BN,),
        in_specs=[
            pl.BlockSpec((_BN, din), lambda i: (i, 0)),
            pl.BlockSpec((din, dout), lambda i: (0, 0)),
            pl.BlockSpec((1, dout), lambda i: (0, 0)),
        ],
        out_specs=pl.BlockSpec((_BN, dout), lambda i: (i, 0)),
        out_shape=jax.ShapeDtypeStruct((n, dout), F32),
    )(x, w, b.reshape(1, dout))


def _embed_body(x_ref, w_ref, b_ref, xp_ref, o_ref, t_ref):
    h = jnp.dot(x_ref[...], w_ref[...], preferred_element_type=F32) + b_ref[...]
    o_ref[...] = h
    t_ref[...] = _pack(h, xp_ref[...])


def _embed(x, w, b, xpad):
    n, din = x.shape
    dout = w.shape[1]
    return pl.pallas_call(
        _embed_body,
        grid=(n // _BN,),
        in_specs=[
            pl.BlockSpec((_BN, din), lambda i: (i, 0)),
            pl.BlockSpec((din, dout), lambda i: (0, 0)),
            pl.BlockSpec((1, dout), lambda i: (0, 0)),
            pl.BlockSpec((_BN, _XP), lambda i: (i, 0)),
        ],
        out_specs=[
            pl.BlockSpec((_BN, dout), lambda i: (i, 0)),
            pl.BlockSpec((_BN, _XG), lambda i: (i, 0)),
        ],
        out_shape=[
            jax.ShapeDtypeStruct((n, dout), F32),
            jax.ShapeDtypeStruct((n, _XG), jnp.int32),
        ],
    )(x, w, b.reshape(1, dout), xpad)


def _edge_body(td_ref, ts_ref, w1_ref,
               b1_ref, w2_ref, b2_ref, wc1_ref, bc1_ref, wc2_ref, bc2_ref,
               m_ref, mc_ref):
    hd, xd = _unpack(td_ref[...])
    hs, xs = _unpack(ts_ref[...])
    diff = xd - xs
    r2 = jnp.sum(diff * diff, axis=1, keepdims=True)
    em = jnp.concatenate([hd, hs, r2], axis=1)
    t = _silu(jnp.dot(em, w1_ref[...], preferred_element_type=F32)
              + b1_ref[...])
    m = _silu(jnp.dot(t, w2_ref[...], preferred_element_type=F32) + b2_ref[...])
    u = _silu(jnp.dot(m, wc1_ref[...], preferred_element_type=F32) + bc1_ref[...])
    cw = jnp.dot(u, wc2_ref[...], preferred_element_type=F32) + bc2_ref[...]
    m_ref[...] = m
    mc_ref[...] = diff * cw


def _edge_tc(td, ts, w1, b1, w2, b2, wc1, bc1, wc2, bc2):
    e = td.shape[0]
    h = w2.shape[1]
    full = lambda r, c: pl.BlockSpec((r, c), lambda i: (0, 0))
    return pl.pallas_call(
        _edge_body,
        grid=(e // _BE,),
        in_specs=[
            pl.BlockSpec((_BE, _XG), lambda i: (i, 0)),
            pl.BlockSpec((_BE, _XG), lambda i: (i, 0)),
            full(w1.shape[0], h), full(1, h),
            full(h, h), full(1, h), full(h, h), full(1, h),
            full(h, 1), full(1, 1),
        ],
        out_specs=[
            pl.BlockSpec((_BE, h), lambda i: (i, 0)),
            pl.BlockSpec((_BE, _XP), lambda i: (i, 0)),
        ],
        out_shape=[
            jax.ShapeDtypeStruct((e, h), F32),
            jax.ShapeDtypeStruct((e, _XP), F32),
        ],
    )(td, ts, w1, b1, w2, b2, wc1, bc1, wc2, bc2)


def _node_body(h_ref, am_ref, am1_ref, ax_ref, ax1_ref, x_ref,
               wn1_ref, bn1_ref,
               wn2_ref, bn2_ref, ho_ref, xo_ref, to_ref):
    aggm = (am_ref[0] + am_ref[1]) + (am1_ref[0] + am1_ref[1])
    aggx = ((ax_ref[0] + ax_ref[1]) + (ax1_ref[0] + ax1_ref[1]))[:, :_XP]
    nm = jnp.concatenate([h_ref[...], aggm], axis=1)
    t = _silu(jnp.dot(nm, wn1_ref[...], preferred_element_type=F32)
              + bn1_ref[...])
    ho = (h_ref[...]
          + jnp.dot(t, wn2_ref[...], preferred_element_type=F32)
          + bn2_ref[...])
    xo = x_ref[...] + aggx / 10.0
    ho_ref[...] = ho
    xo_ref[...] = xo
    to_ref[...] = _pack(ho, xo)


def _node_tc(h, am2, am2b, ax2, ax2b, xpad, wn1, bn1, wn2, bn2):
    n, hd = h.shape
    full = lambda r, c: pl.BlockSpec((r, c), lambda i: (0, 0))
    return pl.pallas_call(
        _node_body,
        grid=(n // _BN,),
        in_specs=[
            pl.BlockSpec((_BN, hd), lambda i: (i, 0)),
            pl.BlockSpec((2, _BN, hd), lambda i: (0, i, 0)),
            pl.BlockSpec((2, _BN, hd), lambda i: (0, i, 0)),
            pl.BlockSpec((2, _BN, _XG), lambda i: (0, i, 0)),
            pl.BlockSpec((2, _BN, _XG), lambda i: (0, i, 0)),
            pl.BlockSpec((_BN, _XP), lambda i: (i, 0)),
            full(2 * hd, hd), full(1, hd), full(hd, hd), full(1, hd),
        ],
        out_specs=[
            pl.BlockSpec((_BN, hd), lambda i: (i, 0)),
            pl.BlockSpec((_BN, _XP), lambda i: (i, 0)),
            pl.BlockSpec((_BN, _XG), lambda i: (i, 0)),
        ],
        out_shape=[
            jax.ShapeDtypeStruct((n, hd), F32),
            jax.ShapeDtypeStruct((n, _XP), F32),
            jax.ShapeDtypeStruct((n, _XG), jnp.int32),
        ],
    )(h, am2, am2b, ax2, ax2b, xpad, wn1, bn1, wn2, bn2)


def _sc_gather(tbl, src2, dst2):
    n, d = tbl.shape
    nch = src2.shape[1]
    e = _NW * nch * _K
    mesh = plsc.VectorSubcoreMesh(core_axis_name="c", subcore_axis_name="s")

    @functools.partial(
        pl.kernel,
        out_type=(
            jax.ShapeDtypeStruct((e, d), jnp.int32),
            jax.ShapeDtypeStruct((e, d), jnp.int32),
        ),
        mesh=mesh,
        scratch_types=[
            pltpu.VMEM((nch, _K), jnp.int32),
            pltpu.VMEM((nch, _K), jnp.int32),
            pltpu.VMEM((2, _K, _XG), jnp.int32),
            pltpu.VMEM((2, _K, _XG), jnp.int32),
            pltpu.SemaphoreType.DMA,
            pltpu.SemaphoreType.DMA,
            pltpu.SemaphoreType.DMA,
            pltpu.SemaphoreType.DMA,
        ],
    )
    def k(t_hbm, src_hbm, dst_hbm, td_hbm, ts_hbm,
          sidx, didx, tdb, tsb, semg0, semg1, semw0, semw1):
        c = lax.axis_index("c")
        s = lax.axis_index("s")
        wid = s * _NC + c
        semg = (semg0, semg1)
        semw = (semw0, semw1)
        pltpu.sync_copy(dst_hbm.at[wid], didx)
        pltpu.sync_copy(src_hbm.at[wid], sidx)

        def gathers(j, b, start):
            g1 = pltpu.make_async_copy(t_hbm.at[didx.at[j]], tdb.at[b],
                                       semg[b])
            g2 = pltpu.make_async_copy(t_hbm.at[sidx.at[j]], tsb.at[b],
                                       semg[b])
            if start:
                g1.start(); g2.start()
            else:
                g1.wait(); g2.wait()

        def writes(j, b, start):
            off = (wid * nch + j) * _K
            w1 = pltpu.make_async_copy(tdb.at[b], td_hbm.at[pl.ds(off, _K)],
                                       semw[b])
            w2 = pltpu.make_async_copy(tsb.at[b], ts_hbm.at[pl.ds(off, _K)],
                                       semw[b])
            if start:
                w1.start(); w2.start()
            else:
                w1.wait(); w2.wait()

        gathers(0, 0, True)

        def loop(i, carry):
            for b in (0, 1):
                j = 2 * i + b
                gathers(j, b, False)

                @pl.when(j + 1 < nch)
                def _next():
                    @pl.when(j >= 1)
                    def _drain():
                        writes(j - 1, 1 - b, False)
                    gathers(j + 1, 1 - b, True)

                writes(j, b, True)
            return carry

        lax.fori_loop(0, nch // 2, loop, 0)
        writes(nch - 2, nch % 2, False)
        writes(nch - 1, (nch - 1) % 2, False)

    return k(tbl, src2, dst2)


def _sc_scatter(vals, dst2, zrows):
    e, dv = vals.shape
    d = zrows.shape[1]
    nch = dst2.shape[1]
    rpt = zrows.shape[0]
    ndr = 10
    n = rpt * ndr
    mesh = plsc.VectorSubcoreMesh(core_axis_name="c", subcore_axis_name="s")

    @functools.partial(
        pl.kernel,
        out_type=jax.ShapeDtypeStruct((_NC, n, d), F32),
        mesh=mesh,
        scratch_types=[
            pltpu.VMEM((nch, _K), jnp.int32),
            pltpu.VMEM((_K, d), F32),
            pltpu.VMEM((_K, _XP), F32),
            pltpu.VMEM_SHARED((n, d), F32),
        ],
    )
    def k(v_hbm, dst_hbm, z_hbm, acc_hbm, didx, vb, cb, acc):
        c = lax.axis_index("c")
        s = lax.axis_index("s")
        wid = s * _NC + c

        @pl.when(s < ndr)
        def _init():
            pltpu.sync_copy(z_hbm, acc.at[pl.ds(s * rpt, rpt)])

        if dv < d:
            pltpu.sync_copy(z_hbm.at[pl.ds(0, _K)], vb)
        plsc.subcore_barrier()
        pltpu.sync_copy(dst_hbm.at[wid], didx)

        def chunk(j, carry):
            off = (wid * nch + j) * _K
            if dv < d:
                pltpu.sync_copy(v_hbm.at[pl.ds(off, _K)], cb)
                for r in range(_K):
                    vb[r, pl.ds(0, dv)] = cb[r]
            else:
                pltpu.sync_copy(v_hbm.at[pl.ds(off, _K)], vb)
            pltpu.sync_copy(vb, acc.at[didx.at[j]], add=True)
            return carry

        lax.fori_loop(0, nch, chunk, 0)
        plsc.subcore_barrier()

        @pl.when(s < ndr)
        def _drain():
            pltpu.sync_copy(acc.at[pl.ds(s * rpt, rpt)],
                            acc_hbm.at[c, pl.ds(s * rpt, rpt)])

    return k(vals, dst2, zrows)


def kernel(feat, coordinate, edge_index, Win, b_in, Wout, b_out,
           We1, be1, We2, be2, Wc1, bc1, Wc2, bc2, Wn1, bn1, Wn2, bn2):
    n, _ = feat.shape
    e = edge_index.shape[1]
    h_dim = Win.shape[1]
    depth = We1.shape[0]

    ea = 153600
    halves = []
    for lo, hi in ((0, ea), (ea, e)):
        eh = hi - lo
        halves.append((
            edge_index[0, lo:hi].reshape(_NW, eh // (_NW * _K), _K),
            edge_index[1, lo:hi].reshape(_NW, eh // (_NW * _K), _K),
        ))
    xpad = jnp.pad(coordinate, ((0, 0), (0, _XP - coordinate.shape[1])))
    z128 = jnp.zeros((n // 10, _XG), F32)

    h, tbl = _embed(feat, Win, b_in, xpad)
    for l in range(depth):
        ew = (
            We1[l],
            be1[l].reshape(1, -1), We2[l], be2[l].reshape(1, -1),
            Wc1[l], bc1[l].reshape(1, -1), Wc2[l], bc2[l].reshape(1, 1),
        )
        g0 = _sc_gather(tbl, *halves[0])
        g1 = _sc_gather(tbl, *halves[1])
        m0, mc0 = _edge_tc(*g0, *ew)
        m1, mc1 = _edge_tc(*g1, *ew)
        am0 = _sc_scatter(m0, halves[0][1], z128)
        ax0 = _sc_scatter(mc0, halves[0][1], z128)
        am1 = _sc_scatter(m1, halves[1][1], z128)
        ax1 = _sc_scatter(mc1, halves[1][1], z128)
        h, xpad, tbl = _node_tc(
            h, am0, am1, ax0, ax1, xpad,
            Wn1[l], bn1[l].reshape(1, -1),
            Wn2[l], bn2[l].reshape(1, -1),
        )
    out = _mmb(h, Wout, b_out)
    return (out, xpad[:, :coordinate.shape[1]])
